# baseline (speedup 1.0000x reference)
import jax
import jax.numpy as jnp
from jax import lax
from jax.experimental import pallas as pl
from jax.experimental.pallas import tpu as pltpu
from jax.experimental.pallas import tpu_sc as plsc

DIM = 64
SCALE = 8.0

NC = 2
NS = 16
NW = NC * NS
SEQ = 200
BATCH = 4096
NEMB = 1_000_000
NTI = 800
TPW = NTI // NW
TSUB = 2
NSTEP = TPW * (8 // TSUB)

NFULL = NEMB // 128
ABASE = NFULL // NW
AREM = NFULL - ABASE * NW
TAIL0 = NFULL * 128


def _tbody(tabT_hbm, tail_hbm, scr_hbm, in_v, out_v, gsem, ssem):
  cid = lax.axis_index("c")
  sid = lax.axis_index("s")
  wid = sid * NC + cid
  nblk = ABASE + jnp.where(wid < AREM, 1, 0)
  start = wid * ABASE + jnp.minimum(wid, AREM)

  iota16 = lax.iota(jnp.int32, 16)
  iotah = iota16 >> 1
  parv = (iota16 & 1) * DIM

  def fire_in(j, b):
    cg = start + j
    pltpu.async_copy(
        tabT_hbm.at[pl.ds(0, DIM), pl.ds(cg * 128, 128)],
        in_v.at[b], gsem.at[b])

  def wait_in(b):
    pltpu.make_async_copy(
        tabT_hbm.at[pl.ds(0, DIM), pl.ds(0, 128)], in_v.at[b], gsem.at[b]
    ).wait()

  def fire_out(j, b):
    cg = start + j
    pltpu.async_copy(
        out_v.at[b],
        scr_hbm.at[pl.ds(cg * 64, 64), pl.ds(0, 128)],
        ssem.at[b])

  def wait_out(b):
    pltpu.make_async_copy(
        out_v.at[b], scr_hbm.at[pl.ds(0, 64), pl.ds(0, 128)], ssem.at[b]
    ).wait()

  def xpose_a(b, nb0):
    bvec = jnp.full((16,), b, jnp.int32)

    @plsc.parallel_loop(0, nb0 * DIM, unroll=8)
    def _(i):
      b0 = (i >> 6) << 4
      f = i & 63
      v = in_v[b, f, pl.ds(b0, 16)] * SCALE
      plsc.store_scatter(
          out_v, [bvec, (b0 >> 1) + iotah, parv + f], v)

  @pl.when(wid == NW - 1)
  def _():
    pltpu.sync_copy(tail_hbm.at[pl.ds(0, DIM), pl.ds(0, 128)],
                    in_v.at[0, pl.ds(0, DIM), pl.ds(0, 128)])
    xpose_a(0, 4)
    pltpu.sync_copy(out_v.at[0, pl.ds(0, 32), pl.ds(0, 128)],
                    scr_hbm.at[pl.ds(TAIL0 // 2, 32), pl.ds(0, 128)])

  fire_in(0, 0)

  @pl.loop(0, ABASE + 1, step=2)
  def _(j):
    for b in range(2):
      jj = j + b

      @pl.when(jj < nblk)
      def _():
        @pl.when(jj + 1 < nblk)
        def _():
          @pl.when(jj >= 1)
          def _():
            wait_out(1 - b)
          fire_in(jj + 1, 1 - b)

        wait_in(b)
        xpose_a(b, 8)
        fire_out(jj, b)

  wait_out(0)
  wait_out(1)


def _body(ids_hbm, tab_hbm, out_hbm, idx_v, sidx_v, rows_v, tile_v,
          gsem, ssem):
  cid = lax.axis_index("c")
  sid = lax.axis_index("s")
  wid = sid * NC + cid

  pltpu.sync_copy(ids_hbm.at[pl.ds(wid * TPW, TPW)], idx_v)

  iota16 = lax.iota(jnp.int32, 16)
  zeros16 = jnp.zeros((16,), jnp.int32)

  def fire_gather(step, b):
    k = step // 4
    q = lax.rem(step, 4)
    for j in range(TSUB):
      for c in range(8):
        sidx_v[b, j, pl.ds(c * 16, 16)] = (
            idx_v[k, q * TSUB + j, pl.ds(c * 16, 16)] >> 1)
      pltpu.async_copy(
          tab_hbm.at[sidx_v.at[b, j]],
          rows_v.at[b, j],
          gsem.at[b],
      )

  def wait_gather(b):
    for j in range(TSUB):
      pltpu.make_async_copy(
          tab_hbm.at[sidx_v.at[0, 0]], rows_v.at[b, j], gsem.at[b]
      ).wait()

  def fire_store(step, b):
    k = step // 4
    q = lax.rem(step, 4)
    ft = wid * TPW + k
    tr = ft // 32
    tc = lax.rem(ft, 32)
    pltpu.async_copy(
        tile_v.at[b],
        out_hbm.at[pl.ds(8 * tr + TSUB * q, TSUB), pl.ds(0, 8),
                   pl.ds(tc, 1), pl.ds(0, 8), pl.ds(0, 128)],
        ssem.at[b],
    )

  def wait_store(b):
    pltpu.make_async_copy(
        tile_v.at[b],
        out_hbm.at[pl.ds(0, TSUB), pl.ds(0, 8), pl.ds(0, 1),
                   pl.ds(0, 8), pl.ds(0, 128)],
        ssem.at[b],
    ).wait()

  def xpose(step, b):
    k = step // 4
    q = lax.rem(step, 4)
    bvec = jnp.full((16,), b, jnp.int32)

    @plsc.parallel_loop(0, TSUB * 512, unroll=8)
    def _(i):
      ti = i >> 9
      b0 = ((i >> 6) & 7) << 4
      f = i & 63
      r = f >> 3
      fr = f & 7
      par = (idx_v[k, q * TSUB + ti, pl.ds(b0, 16)] & 1) * DIM
      v = plsc.load_gather(
          rows_v,
          [bvec, zeros16 + ti, b0 + iota16, par + f],
      )
      tile_v[b, ti, r, 0, fr, pl.ds(b0, 16)] = v

  fire_gather(0, 0)

  @pl.loop(0, NSTEP, step=2)
  def _(s):
    for b in range(2):
      ss = s + b

      @pl.when(ss + 1 < NSTEP)
      def _():
        @pl.when(ss >= 1)
        def _():
          wait_store(1 - b)
        fire_gather(ss + 1, 1 - b)

      wait_gather(b)
      xpose(ss, b)
      fire_store(ss, b)

  wait_store(0)
  wait_store(1)


@jax.jit
def _embed(ids_in, tabT, tail):
  mesh = plsc.VectorSubcoreMesh(core_axis_name="c", subcore_axis_name="s")
  fmt = pl.kernel(
      _tbody,
      out_type=jax.ShapeDtypeStruct((NEMB // 2, 128), jnp.float32),
      mesh=mesh,
      scratch_types=[
          pltpu.VMEM((2, DIM, 128), jnp.float32),
          pltpu.VMEM((2, DIM, 128), jnp.float32),
          pltpu.SemaphoreType.DMA((2,)),
          pltpu.SemaphoreType.DMA((2,)),
      ],
      compiler_params=pltpu.CompilerParams(
          use_tc_tiling_on_sc=True, needs_layout_passes=False),
  )
  tab2 = fmt(tabT, tail)
  run = pl.kernel(
      _body,
      out_type=jax.ShapeDtypeStruct((SEQ, 8, 32, 8, 128), jnp.float32),
      mesh=mesh,
      scratch_types=[
          pltpu.VMEM((TPW, 8, 128), jnp.int32),
          pltpu.VMEM((2, TSUB, 128), jnp.int32),
          pltpu.VMEM((2, TSUB, 128, 128), jnp.float32),
          pltpu.VMEM((2, TSUB, 8, 1, 8, 128), jnp.float32),
          pltpu.SemaphoreType.DMA((2,)),
          pltpu.SemaphoreType.DMA((2,)),
      ],
      compiler_params=pltpu.CompilerParams(
          use_tc_tiling_on_sc=True, needs_layout_passes=False),
  )
  return run(ids_in, tab2)


def kernel(token_ids, tok_embedding):
  ids_in = (jnp.transpose(token_ids).reshape(25, 8, 32, 128)
            .transpose(0, 2, 1, 3).reshape(NTI, 8, 128)
            .astype(jnp.int32))
  tabT = jnp.transpose(tok_embedding)
  tail = jnp.pad(tok_embedding[TAIL0:, :].T,
                 ((0, 0), (0, 128 - (NEMB - TAIL0))))
  o = _embed(ids_in, tabT, tail)
  return o.transpose(2, 4, 0, 1, 3).reshape(BATCH, SEQ, DIM)


# speedup vs baseline: 2.8728x; 2.8728x over previous
import jax
import jax.numpy as jnp
from jax import lax
from jax.experimental import pallas as pl
from jax.experimental.pallas import tpu as pltpu
from jax.experimental.pallas import tpu_sc as plsc

DIM = 64
SCALE = 8.0

NC = 2
NS = 16
NW = NC * NS
SEQ = 200
BATCH = 4096
NEMB = 1_000_000
NTI = 800
TPW = NTI // NW
TSUB = 2
NSTEP = TPW * (8 // TSUB)

NFULL = NEMB // 128
ABASE = NFULL // NW
AREM = NFULL - ABASE * NW
TAIL0 = NFULL * 128


def _tbody(tabT_hbm, tail_hbm, scr_hbm, in_v, out_v, gsem, ssem):
  cid = lax.axis_index("c")
  sid = lax.axis_index("s")
  wid = sid * NC + cid
  nblk = ABASE + jnp.where(wid < AREM, 1, 0)
  start = wid * ABASE + jnp.minimum(wid, AREM)

  iota16 = lax.iota(jnp.int32, 16)

  def fire_in(j, b):
    cg = start + j
    pltpu.async_copy(
        tabT_hbm.at[pl.ds(0, DIM), pl.ds(cg * 128, 128)],
        in_v.at[b], gsem.at[b])

  def wait_in(b):
    pltpu.make_async_copy(
        tabT_hbm.at[pl.ds(0, DIM), pl.ds(0, 128)], in_v.at[b], gsem.at[b]
    ).wait()

  def fire_out(j, b):
    cg = start + j
    pltpu.async_copy(
        out_v.at[b], scr_hbm.at[pl.ds(cg * 128, 128), pl.ds(0, 128)],
        ssem.at[b])

  def wait_out(b):
    pltpu.make_async_copy(
        out_v.at[b], scr_hbm.at[pl.ds(0, 128), pl.ds(0, 128)], ssem.at[b]
    ).wait()

  def xpose_a(b, nid0bits):
    bvec = jnp.full((16,), b, jnp.int32)
    nid0 = 1 << nid0bits

    @plsc.parallel_loop(0, 16 * 4 * nid0, unroll=4)
    def _(i):
      d = i >> (2 + nid0bits)
      f0 = ((i >> nid0bits) & 3) * 16
      id0 = (i & (nid0 - 1)) * 16
      dmask = (iota16 + d) & 15
      fvec = f0 + iota16
      idvec = id0 + dmask
      v = plsc.load_gather(in_v, [bvec, fvec, idvec])
      plsc.store_scatter(out_v, [bvec, idvec, fvec], v * SCALE)

  @pl.when(wid == NW - 1)
  def _():
    pltpu.sync_copy(tail_hbm.at[pl.ds(0, DIM), pl.ds(0, 128)], in_v.at[0])
    xpose_a(0, 2)
    pltpu.sync_copy(out_v.at[0, pl.ds(0, DIM)],
                    scr_hbm.at[pl.ds(TAIL0, DIM), pl.ds(0, 128)])

  fire_in(0, 0)

  @pl.loop(0, ABASE + 1, step=2)
  def _(j):
    for b in range(2):
      jj = j + b

      @pl.when(jj < nblk)
      def _():
        @pl.when(jj + 1 < nblk)
        def _():
          @pl.when(jj >= 1)
          def _():
            wait_out(1 - b)
          fire_in(jj + 1, 1 - b)

        wait_in(b)
        xpose_a(b, 3)
        fire_out(jj, b)

  wait_out(0)
  wait_out(1)


def _body(ids_hbm, tab_hbm, out_hbm, idx_v, rows_v, tile_v, gsem, ssem):
  cid = lax.axis_index("c")
  sid = lax.axis_index("s")
  wid = sid * NC + cid

  pltpu.sync_copy(ids_hbm.at[pl.ds(wid * TPW, TPW)], idx_v)

  iota16 = lax.iota(jnp.int32, 16)
  zeros16 = jnp.zeros((16,), jnp.int32)

  def fire_gather(step, b):
    k = step // 4
    q = lax.rem(step, 4)
    for j in range(TSUB):
      pltpu.async_copy(
          tab_hbm.at[idx_v.at[k, q * TSUB + j]],
          rows_v.at[b, j],
          gsem.at[b],
      )

  def wait_gather(b):
    for j in range(TSUB):
      pltpu.make_async_copy(
          tab_hbm.at[idx_v.at[0, 0]], rows_v.at[b, j], gsem.at[b]
      ).wait()

  def fire_store(step, b):
    k = step // 4
    q = lax.rem(step, 4)
    ft = wid * TPW + k
    tr = ft // 32
    tc = lax.rem(ft, 32)
    pltpu.async_copy(
        tile_v.at[b],
        out_hbm.at[pl.ds(8 * tr + TSUB * q, TSUB), pl.ds(0, 8),
                   pl.ds(tc, 1), pl.ds(0, 8), pl.ds(0, 128)],
        ssem.at[b],
    )

  def wait_store(b):
    pltpu.make_async_copy(
        tile_v.at[b],
        out_hbm.at[pl.ds(0, TSUB), pl.ds(0, 8), pl.ds(0, 1),
                   pl.ds(0, 8), pl.ds(0, 128)],
        ssem.at[b],
    ).wait()

  def xpose(b):
    bvec = jnp.full((16,), b, jnp.int32)

    @plsc.parallel_loop(0, TSUB * 512, unroll=4)
    def _(i):
      ti = i >> 9
      d = (i >> 5) & 15
      f0 = ((i >> 3) & 3) * 16
      row0 = (i & 7) * 16
      dmask = (iota16 + d) & 15
      fvec = f0 + iota16
      rowvec = row0 + dmask
      tivec = zeros16 + ti
      v = plsc.load_gather(rows_v, [bvec, tivec, rowvec, fvec])
      plsc.store_scatter(
          tile_v,
          [bvec, tivec, fvec >> 3, zeros16, fvec & 7, rowvec],
          v)

  fire_gather(0, 0)

  @pl.loop(0, NSTEP, step=2)
  def _(s):
    for b in range(2):
      ss = s + b

      @pl.when(ss + 1 < NSTEP)
      def _():
        @pl.when(ss >= 1)
        def _():
          wait_store(1 - b)
        fire_gather(ss + 1, 1 - b)

      wait_gather(b)
      xpose(b)
      fire_store(ss, b)

  wait_store(0)
  wait_store(1)


@jax.jit
def _embed(ids_in, tabT, tail):
  mesh = plsc.VectorSubcoreMesh(core_axis_name="c", subcore_axis_name="s")
  fmt = pl.kernel(
      _tbody,
      out_type=jax.ShapeDtypeStruct((NEMB, 128), jnp.float32),
      mesh=mesh,
      scratch_types=[
          pltpu.VMEM((2, DIM, 128), jnp.float32),
          pltpu.VMEM((2, 128, 128), jnp.float32),
          pltpu.SemaphoreType.DMA((2,)),
          pltpu.SemaphoreType.DMA((2,)),
      ],
      compiler_params=pltpu.CompilerParams(
          use_tc_tiling_on_sc=True, needs_layout_passes=False),
  )
  tab2 = fmt(tabT, tail)
  run = pl.kernel(
      _body,
      out_type=jax.ShapeDtypeStruct((SEQ, 8, 32, 8, 128), jnp.float32),
      mesh=mesh,
      scratch_types=[
          pltpu.VMEM((TPW, 8, 128), jnp.int32),
          pltpu.VMEM((2, TSUB, 128, 128), jnp.float32),
          pltpu.VMEM((2, TSUB, 8, 1, 8, 128), jnp.float32),
          pltpu.SemaphoreType.DMA((2,)),
          pltpu.SemaphoreType.DMA((2,)),
      ],
      compiler_params=pltpu.CompilerParams(
          use_tc_tiling_on_sc=True, needs_layout_passes=False),
  )
  return run(ids_in, tab2)


def kernel(token_ids, tok_embedding):
  ids_in = (jnp.transpose(token_ids).reshape(25, 8, 32, 128)
            .transpose(0, 2, 1, 3).reshape(NTI, 8, 128)
            .astype(jnp.int32))
  tabT = jnp.transpose(tok_embedding)
  tail = jnp.pad(tok_embedding[TAIL0:, :].T,
                 ((0, 0), (0, 128 - (NEMB - TAIL0))))
  o = _embed(ids_in, tabT, tail)
  return o.transpose(2, 4, 0, 1, 3).reshape(BATCH, SEQ, DIM)


# speedup vs baseline: 2.9503x; 1.0270x over previous
import jax
import jax.numpy as jnp
from jax import lax
from jax.experimental import pallas as pl
from jax.experimental.pallas import tpu as pltpu
from jax.experimental.pallas import tpu_sc as plsc

DIM = 64
SCALE = 8.0

NC = 2
NS = 16
NW = NC * NS
SEQ = 200
BATCH = 4096
NEMB = 1_000_000
NTI = 800
TPW = NTI // NW
TSUB = 2
NSTEP = TPW * (8 // TSUB)

NFULL = NEMB // 128
ABASE = NFULL // NW
AREM = NFULL - ABASE * NW
TAIL0 = NFULL * 128


def _tbody(tabT_hbm, tail_hbm, scr_hbm, in_v, out_v, gsem, ssem):
  cid = lax.axis_index("c")
  sid = lax.axis_index("s")
  wid = sid * NC + cid
  nblk = ABASE + jnp.where(wid < AREM, 1, 0)
  start = wid * ABASE + jnp.minimum(wid, AREM)

  iota16 = lax.iota(jnp.int32, 16)

  def fire_in(j, b):
    cg = start + j
    pltpu.async_copy(
        tabT_hbm.at[pl.ds(0, DIM), pl.ds(cg * 128, 128)],
        in_v.at[b], gsem.at[b])

  def wait_in(b):
    pltpu.make_async_copy(
        tabT_hbm.at[pl.ds(0, DIM), pl.ds(0, 128)], in_v.at[b], gsem.at[b]
    ).wait()

  def fire_out(j, b):
    cg = start + j
    pltpu.async_copy(
        out_v.at[b], scr_hbm.at[pl.ds(cg * 128, 128), pl.ds(0, 128)],
        ssem.at[b])

  def wait_out(b):
    pltpu.make_async_copy(
        out_v.at[b], scr_hbm.at[pl.ds(0, 128), pl.ds(0, 128)], ssem.at[b]
    ).wait()

  def xpose_a(b, nid0bits):
    bvec = jnp.full((16,), b, jnp.int32)
    nid0 = 1 << nid0bits

    @plsc.parallel_loop(0, 16 * 4 * nid0, unroll=8)
    def _(i):
      d = i >> (2 + nid0bits)
      f0 = ((i >> nid0bits) & 3) * 16
      id0 = (i & (nid0 - 1)) * 16
      dmask = (iota16 + d) & 15
      fvec = f0 + iota16
      idvec = id0 + dmask
      v = plsc.load_gather(in_v, [bvec, fvec, idvec])
      plsc.store_scatter(out_v, [bvec, idvec, fvec], v * SCALE)

  @pl.when(wid == NW - 1)
  def _():
    pltpu.sync_copy(tail_hbm.at[pl.ds(0, DIM), pl.ds(0, 128)], in_v.at[0])
    xpose_a(0, 2)
    pltpu.sync_copy(out_v.at[0, pl.ds(0, DIM)],
                    scr_hbm.at[pl.ds(TAIL0, DIM), pl.ds(0, 128)])

  fire_in(0, 0)

  @pl.loop(0, ABASE + 1, step=2)
  def _(j):
    for b in range(2):
      jj = j + b

      @pl.when(jj < nblk)
      def _():
        @pl.when(jj + 1 < nblk)
        def _():
          @pl.when(jj >= 1)
          def _():
            wait_out(1 - b)
          fire_in(jj + 1, 1 - b)

        wait_in(b)
        xpose_a(b, 3)
        fire_out(jj, b)

  wait_out(0)
  wait_out(1)


def _body(ids_hbm, tab_hbm, out_hbm, idx_v, rows_v, tile_v, gsem, ssem):
  cid = lax.axis_index("c")
  sid = lax.axis_index("s")
  wid = sid * NC + cid

  pltpu.sync_copy(ids_hbm.at[pl.ds(wid * TPW, TPW)], idx_v)

  iota16 = lax.iota(jnp.int32, 16)
  zeros16 = jnp.zeros((16,), jnp.int32)

  def fire_gather(step, b):
    k = step // 4
    q = lax.rem(step, 4)
    for j in range(TSUB):
      pltpu.async_copy(
          tab_hbm.at[idx_v.at[k, q * TSUB + j]],
          rows_v.at[b, j],
          gsem.at[b],
      )

  def wait_gather(b):
    for j in range(TSUB):
      pltpu.make_async_copy(
          tab_hbm.at[idx_v.at[0, 0]], rows_v.at[b, j], gsem.at[b]
      ).wait()

  def fire_store(step, b):
    k = step // 4
    q = lax.rem(step, 4)
    ft = wid * TPW + k
    tr = ft // 32
    tc = lax.rem(ft, 32)
    pltpu.async_copy(
        tile_v.at[b],
        out_hbm.at[pl.ds(8 * tr + TSUB * q, TSUB), pl.ds(0, 8),
                   pl.ds(tc, 1), pl.ds(0, 8), pl.ds(0, 128)],
        ssem.at[b],
    )

  def wait_store(b):
    pltpu.make_async_copy(
        tile_v.at[b],
        out_hbm.at[pl.ds(0, TSUB), pl.ds(0, 8), pl.ds(0, 1),
                   pl.ds(0, 8), pl.ds(0, 128)],
        ssem.at[b],
    ).wait()

  def xpose(b):
    bvec = jnp.full((16,), b, jnp.int32)

    @plsc.parallel_loop(0, TSUB * 512, unroll=8)
    def _(i):
      ti = i >> 9
      d = (i >> 5) & 15
      f0 = ((i >> 3) & 3) * 16
      row0 = (i & 7) * 16
      dmask = (iota16 + d) & 15
      fvec = f0 + iota16
      rowvec = row0 + dmask
      tivec = zeros16 + ti
      v = plsc.load_gather(rows_v, [bvec, tivec, rowvec, fvec])
      plsc.store_scatter(
          tile_v,
          [bvec, tivec, fvec >> 3, zeros16, fvec & 7, rowvec],
          v)

  fire_gather(0, 0)

  @pl.loop(0, NSTEP, step=2)
  def _(s):
    for b in range(2):
      ss = s + b

      @pl.when(ss + 1 < NSTEP)
      def _():
        @pl.when(ss >= 1)
        def _():
          wait_store(1 - b)
        fire_gather(ss + 1, 1 - b)

      wait_gather(b)
      xpose(b)
      fire_store(ss, b)

  wait_store(0)
  wait_store(1)


@jax.jit
def _embed(ids_in, tabT, tail):
  mesh = plsc.VectorSubcoreMesh(core_axis_name="c", subcore_axis_name="s")
  fmt = pl.kernel(
      _tbody,
      out_type=jax.ShapeDtypeStruct((NEMB, 128), jnp.float32),
      mesh=mesh,
      scratch_types=[
          pltpu.VMEM((2, DIM, 128), jnp.float32),
          pltpu.VMEM((2, 128, 128), jnp.float32),
          pltpu.SemaphoreType.DMA((2,)),
          pltpu.SemaphoreType.DMA((2,)),
      ],
      compiler_params=pltpu.CompilerParams(
          use_tc_tiling_on_sc=True, needs_layout_passes=False),
  )
  tab2 = fmt(tabT, tail)
  run = pl.kernel(
      _body,
      out_type=jax.ShapeDtypeStruct((SEQ, 8, 32, 8, 128), jnp.float32),
      mesh=mesh,
      scratch_types=[
          pltpu.VMEM((TPW, 8, 128), jnp.int32),
          pltpu.VMEM((2, TSUB, 128, 128), jnp.float32),
          pltpu.VMEM((2, TSUB, 8, 1, 8, 128), jnp.float32),
          pltpu.SemaphoreType.DMA((2,)),
          pltpu.SemaphoreType.DMA((2,)),
      ],
      compiler_params=pltpu.CompilerParams(
          use_tc_tiling_on_sc=True, needs_layout_passes=False),
  )
  return run(ids_in, tab2)


def kernel(token_ids, tok_embedding):
  ids_in = (jnp.transpose(token_ids).reshape(25, 8, 32, 128)
            .transpose(0, 2, 1, 3).reshape(NTI, 8, 128)
            .astype(jnp.int32))
  tabT = jnp.transpose(tok_embedding)
  tail = jnp.pad(tok_embedding[TAIL0:, :].T,
                 ((0, 0), (0, 128 - (NEMB - TAIL0))))
  o = _embed(ids_in, tabT, tail)
  return o.transpose(2, 4, 0, 1, 3).reshape(BATCH, SEQ, DIM)


# speedup vs baseline: 3.5485x; 1.2028x over previous
import jax
import jax.numpy as jnp
from jax import lax
from jax.experimental import pallas as pl
from jax.experimental.pallas import tpu as pltpu
from jax.experimental.pallas import tpu_sc as plsc

DIM = 64
SCALE = 8.0

NC = 2
NS = 16
NW = NC * NS
SEQ = 200
BATCH = 4096
NEMB = 1_000_000
NTI = 800
TPW = NTI // NW
TSUB = 2
NSTEP = TPW * (8 // TSUB)

NFULL = NEMB // 128
ABASE = NFULL // NW
AREM = NFULL - ABASE * NW
TAIL0 = NFULL * 128


def _tbody(tabT_hbm, tail_hbm, scr_hbm, in_v, out_v, gsem, ssem):
  cid = lax.axis_index("c")
  sid = lax.axis_index("s")
  wid = sid * NC + cid
  nblk = ABASE + jnp.where(wid < AREM, 1, 0)
  start = wid * ABASE + jnp.minimum(wid, AREM)

  iota16 = lax.iota(jnp.int32, 16)

  def fire_in(j, b):
    cg = start + j
    pltpu.async_copy(
        tabT_hbm.at[pl.ds(0, DIM), pl.ds(cg * 128, 128)],
        in_v.at[b], gsem.at[b])

  def wait_in(b):
    pltpu.make_async_copy(
        tabT_hbm.at[pl.ds(0, DIM), pl.ds(0, 128)], in_v.at[b], gsem.at[b]
    ).wait()

  def fire_out(j, b):
    cg = start + j
    pltpu.async_copy(
        out_v.at[b], scr_hbm.at[pl.ds(cg * 64, 64), pl.ds(0, 128)],
        ssem.at[b])

  def wait_out(b):
    pltpu.make_async_copy(
        out_v.at[b], scr_hbm.at[pl.ds(0, 64), pl.ds(0, 128)], ssem.at[b]
    ).wait()

  def xpose_a(b, nid0bits):
    bvec = jnp.full((16,), b, jnp.int32)
    nid0 = 1 << nid0bits

    @plsc.parallel_loop(0, 16 * 4 * nid0, unroll=8)
    def _(i):
      d = i >> (2 + nid0bits)
      f0 = ((i >> nid0bits) & 3) * 16
      id0 = (i & (nid0 - 1)) * 16
      dmask = (iota16 + d) & 15
      fvec = f0 + iota16
      pairvec = (id0 >> 1) + (dmask >> 1)
      colvec = (dmask & 1) * DIM + fvec
      v = plsc.load_gather(in_v, [bvec, fvec, id0 + dmask])
      plsc.store_scatter(out_v, [bvec, pairvec, colvec], v * SCALE)

  @pl.when(wid == NW - 1)
  def _():
    pltpu.sync_copy(tail_hbm.at[pl.ds(0, DIM), pl.ds(0, 128)], in_v.at[0])
    xpose_a(0, 2)
    pltpu.sync_copy(out_v.at[0, pl.ds(0, 32)],
                    scr_hbm.at[pl.ds(TAIL0 // 2, 32), pl.ds(0, 128)])

  fire_in(0, 0)

  @pl.loop(0, ABASE + 1, step=2)
  def _(j):
    for b in range(2):
      jj = j + b

      @pl.when(jj < nblk)
      def _():
        @pl.when(jj + 1 < nblk)
        def _():
          @pl.when(jj >= 1)
          def _():
            wait_out(1 - b)
          fire_in(jj + 1, 1 - b)

        wait_in(b)
        xpose_a(b, 3)
        fire_out(jj, b)

  wait_out(0)
  wait_out(1)


def _body(ids_hbm, tab_hbm, out_hbm, idx_v, rows_v, tile_v, gsem, ssem):
  cid = lax.axis_index("c")
  sid = lax.axis_index("s")
  wid = sid * NC + cid

  pltpu.sync_copy(ids_hbm.at[pl.ds(wid * TPW, TPW)], idx_v)

  iota16 = lax.iota(jnp.int32, 16)
  zeros16 = jnp.zeros((16,), jnp.int32)

  def fire_gather(step, b):
    k = step // 4
    q = lax.rem(step, 4)
    for j in range(TSUB):
      pltpu.async_copy(
          tab_hbm.at[idx_v.at[k, q * TSUB + j]],
          rows_v.at[b, j],
          gsem.at[b],
      )

  def wait_gather(b):
    for j in range(TSUB):
      pltpu.make_async_copy(
          tab_hbm.at[idx_v.at[0, 0]], rows_v.at[b, j], gsem.at[b]
      ).wait()

  def fire_store(step, b):
    k = step // 4
    q = lax.rem(step, 4)
    ft = wid * TPW + k
    tr = ft // 32
    tc = lax.rem(ft, 32)
    pltpu.async_copy(
        tile_v.at[b],
        out_hbm.at[pl.ds(8 * tr + TSUB * q, TSUB), pl.ds(0, 8),
                   pl.ds(tc, 1), pl.ds(0, 8), pl.ds(0, 128)],
        ssem.at[b],
    )

  def wait_store(b):
    pltpu.make_async_copy(
        tile_v.at[b],
        out_hbm.at[pl.ds(0, TSUB), pl.ds(0, 8), pl.ds(0, 1),
                   pl.ds(0, 8), pl.ds(0, 128)],
        ssem.at[b],
    ).wait()

  def xpose(b):
    bvec = jnp.full((16,), b, jnp.int32)

    @plsc.parallel_loop(0, TSUB * 512, unroll=8)
    def _(i):
      ti = i >> 9
      d = (i >> 5) & 15
      f0 = ((i >> 3) & 3) * 16
      row0 = (i & 7) * 16
      dmask = (iota16 + d) & 15
      fvec = f0 + iota16
      rowvec = row0 + dmask
      tivec = zeros16 + ti
      v = plsc.load_gather(rows_v, [bvec, tivec, rowvec, fvec])
      plsc.store_scatter(
          tile_v,
          [bvec, tivec, fvec >> 3, zeros16, fvec & 7, rowvec],
          v)

  fire_gather(0, 0)

  @pl.loop(0, NSTEP, step=2)
  def _(s):
    for b in range(2):
      ss = s + b

      @pl.when(ss + 1 < NSTEP)
      def _():
        @pl.when(ss >= 1)
        def _():
          wait_store(1 - b)
        fire_gather(ss + 1, 1 - b)

      wait_gather(b)
      xpose(b)
      fire_store(ss, b)

  wait_store(0)
  wait_store(1)


@jax.jit
def _embed(ids_in, tabT, tail):
  mesh = plsc.VectorSubcoreMesh(core_axis_name="c", subcore_axis_name="s")
  fmt = pl.kernel(
      _tbody,
      out_type=jax.ShapeDtypeStruct((NEMB // 2, 128), jnp.float32),
      mesh=mesh,
      scratch_types=[
          pltpu.VMEM((2, DIM, 128), jnp.float32),
          pltpu.VMEM((2, DIM, 128), jnp.float32),
          pltpu.SemaphoreType.DMA((2,)),
          pltpu.SemaphoreType.DMA((2,)),
      ],
      compiler_params=pltpu.CompilerParams(
          use_tc_tiling_on_sc=True, needs_layout_passes=False),
  )
  tab2 = fmt(tabT, tail).reshape(NEMB, DIM)
  run = pl.kernel(
      _body,
      out_type=jax.ShapeDtypeStruct((SEQ, 8, 32, 8, 128), jnp.float32),
      mesh=mesh,
      scratch_types=[
          pltpu.VMEM((TPW, 8, 128), jnp.int32),
          pltpu.VMEM((2, TSUB, 128, DIM), jnp.float32),
          pltpu.VMEM((2, TSUB, 8, 1, 8, 128), jnp.float32),
          pltpu.SemaphoreType.DMA((2,)),
          pltpu.SemaphoreType.DMA((2,)),
      ],
      compiler_params=pltpu.CompilerParams(
          use_tc_tiling_on_sc=False, needs_layout_passes=False),
  )
  return run(ids_in, tab2)


def kernel(token_ids, tok_embedding):
  ids_in = (jnp.transpose(token_ids).reshape(25, 8, 32, 128)
            .transpose(0, 2, 1, 3).reshape(NTI, 8, 128)
            .astype(jnp.int32))
  tabT = jnp.transpose(tok_embedding)
  tail = jnp.pad(tok_embedding[TAIL0:, :].T,
                 ((0, 0), (0, 128 - (NEMB - TAIL0))))
  o = _embed(ids_in, tabT, tail)
  return o.transpose(2, 4, 0, 1, 3).reshape(BATCH, SEQ, DIM)


# speedup vs baseline: 3.8729x; 1.0914x over previous
import jax
import jax.numpy as jnp
from jax import lax
from jax.experimental import pallas as pl
from jax.experimental.pallas import tpu as pltpu
from jax.experimental.pallas import tpu_sc as plsc

DIM = 64
SCALE = 8.0

NC = 2
NS = 16
NW = NC * NS
SEQ = 200
BATCH = 4096
NEMB = 1_000_000
NTI = 800
TPW = NTI // NW
TSUB = 2
NSTEP = TPW * (8 // TSUB)

NFULL = NEMB // 128
ABASE = NFULL // NW
AREM = NFULL - ABASE * NW
TAIL0 = NFULL * 128


def _tbody(tabT_hbm, tail_hbm, scr_hbm, in_v, out_v, gsem, ssem):
  cid = lax.axis_index("c")
  sid = lax.axis_index("s")
  wid = sid * NC + cid
  nblk = ABASE + jnp.where(wid < AREM, 1, 0)
  start = wid * ABASE + jnp.minimum(wid, AREM)

  iota16 = lax.iota(jnp.int32, 16)

  def fire_in(j, b):
    cg = start + j
    pltpu.async_copy(
        tabT_hbm.at[pl.ds(0, DIM), pl.ds(cg * 128, 128)],
        in_v.at[b], gsem.at[b])

  def wait_in(b):
    pltpu.make_async_copy(
        tabT_hbm.at[pl.ds(0, DIM), pl.ds(0, 128)], in_v.at[b], gsem.at[b]
    ).wait()

  def fire_out(j, b):
    cg = start + j
    pltpu.async_copy(
        out_v.at[b], scr_hbm.at[pl.ds(cg * 64, 64), pl.ds(0, 128)],
        ssem.at[b])

  def wait_out(b):
    pltpu.make_async_copy(
        out_v.at[b], scr_hbm.at[pl.ds(0, 64), pl.ds(0, 128)], ssem.at[b]
    ).wait()

  def xpose_a(b, nid0bits):
    bvec = jnp.full((16,), b, jnp.int32)
    nid0 = 1 << nid0bits

    @plsc.parallel_loop(0, 16 * 4 * nid0, unroll=16)
    def _(i):
      d = i >> (2 + nid0bits)
      f0 = ((i >> nid0bits) & 3) * 16
      id0 = (i & (nid0 - 1)) * 16
      dmask = (iota16 + d) & 15
      fvec = f0 + iota16
      pairvec = (id0 >> 1) + (dmask >> 1)
      colvec = (dmask & 1) * DIM + fvec
      v = plsc.load_gather(in_v, [bvec, fvec, id0 + dmask])
      plsc.store_scatter(out_v, [bvec, pairvec, colvec], v * SCALE)

  @pl.when(wid == NW - 1)
  def _():
    pltpu.sync_copy(tail_hbm.at[pl.ds(0, DIM), pl.ds(0, 128)], in_v.at[0])
    xpose_a(0, 2)
    pltpu.sync_copy(out_v.at[0, pl.ds(0, 32)],
                    scr_hbm.at[pl.ds(TAIL0 // 2, 32), pl.ds(0, 128)])

  fire_in(0, 0)

  @pl.loop(0, ABASE + 1, step=2)
  def _(j):
    for b in range(2):
      jj = j + b

      @pl.when(jj < nblk)
      def _():
        @pl.when(jj + 1 < nblk)
        def _():
          @pl.when(jj >= 1)
          def _():
            wait_out(1 - b)
          fire_in(jj + 1, 1 - b)

        wait_in(b)
        xpose_a(b, 3)
        fire_out(jj, b)

  wait_out(0)
  wait_out(1)


def _body(ids_hbm, tab_hbm, out_hbm, idx_v, rows_v, tile_v, gsem, ssem):
  cid = lax.axis_index("c")
  sid = lax.axis_index("s")
  wid = sid * NC + cid

  pltpu.sync_copy(ids_hbm.at[pl.ds(wid * TPW, TPW)], idx_v)

  iota16 = lax.iota(jnp.int32, 16)
  zeros16 = jnp.zeros((16,), jnp.int32)

  def fire_gather(step, b):
    k = step // 4
    q = lax.rem(step, 4)
    for j in range(TSUB):
      pltpu.async_copy(
          tab_hbm.at[idx_v.at[k, q * TSUB + j]],
          rows_v.at[b, j],
          gsem.at[b],
      )

  def wait_gather(b):
    for j in range(TSUB):
      pltpu.make_async_copy(
          tab_hbm.at[idx_v.at[0, 0]], rows_v.at[b, j], gsem.at[b]
      ).wait()

  def fire_store(step, b):
    k = step // 4
    q = lax.rem(step, 4)
    ft = wid * TPW + k
    tr = ft // 32
    tc = lax.rem(ft, 32)
    pltpu.async_copy(
        tile_v.at[b],
        out_hbm.at[pl.ds(8 * tr + TSUB * q, TSUB), pl.ds(0, 8),
                   pl.ds(tc, 1), pl.ds(0, 8), pl.ds(0, 128)],
        ssem.at[b],
    )

  def wait_store(b):
    pltpu.make_async_copy(
        tile_v.at[b],
        out_hbm.at[pl.ds(0, TSUB), pl.ds(0, 8), pl.ds(0, 1),
                   pl.ds(0, 8), pl.ds(0, 128)],
        ssem.at[b],
    ).wait()

  def xpose(b):
    bvec = jnp.full((16,), b, jnp.int32)

    @plsc.parallel_loop(0, TSUB * 512, unroll=16)
    def _(i):
      ti = i >> 9
      d = (i >> 5) & 15
      f0 = ((i >> 3) & 3) * 16
      row0 = (i & 7) * 16
      dmask = (iota16 + d) & 15
      fvec = f0 + iota16
      rowvec = row0 + dmask
      tivec = zeros16 + ti
      v = plsc.load_gather(rows_v, [bvec, tivec, rowvec, fvec])
      plsc.store_scatter(
          tile_v,
          [bvec, tivec, fvec >> 3, zeros16, fvec & 7, rowvec],
          v)

  fire_gather(0, 0)

  @pl.loop(0, NSTEP, step=2)
  def _(s):
    for b in range(2):
      ss = s + b

      @pl.when(ss + 1 < NSTEP)
      def _():
        @pl.when(ss >= 1)
        def _():
          wait_store(1 - b)
        fire_gather(ss + 1, 1 - b)

      wait_gather(b)
      xpose(b)
      fire_store(ss, b)

  wait_store(0)
  wait_store(1)


@jax.jit
def _embed(ids_in, tabT, tail):
  mesh = plsc.VectorSubcoreMesh(core_axis_name="c", subcore_axis_name="s")
  fmt = pl.kernel(
      _tbody,
      out_type=jax.ShapeDtypeStruct((NEMB // 2, 128), jnp.float32),
      mesh=mesh,
      scratch_types=[
          pltpu.VMEM((2, DIM, 128), jnp.float32),
          pltpu.VMEM((2, DIM, 128), jnp.float32),
          pltpu.SemaphoreType.DMA((2,)),
          pltpu.SemaphoreType.DMA((2,)),
      ],
      compiler_params=pltpu.CompilerParams(
          use_tc_tiling_on_sc=True, needs_layout_passes=False),
  )
  tab2 = fmt(tabT, tail).reshape(NEMB, DIM)
  run = pl.kernel(
      _body,
      out_type=jax.ShapeDtypeStruct((SEQ, 8, 32, 8, 128), jnp.float32),
      mesh=mesh,
      scratch_types=[
          pltpu.VMEM((TPW, 8, 128), jnp.int32),
          pltpu.VMEM((2, TSUB, 128, DIM), jnp.float32),
          pltpu.VMEM((2, TSUB, 8, 1, 8, 128), jnp.float32),
          pltpu.SemaphoreType.DMA((2,)),
          pltpu.SemaphoreType.DMA((2,)),
      ],
      compiler_params=pltpu.CompilerParams(
          use_tc_tiling_on_sc=False, needs_layout_passes=False),
  )
  return run(ids_in, tab2)


def kernel(token_ids, tok_embedding):
  ids_in = (jnp.transpose(token_ids).reshape(25, 8, 32, 128)
            .transpose(0, 2, 1, 3).reshape(NTI, 8, 128)
            .astype(jnp.int32))
  tabT = jnp.transpose(tok_embedding)
  tail = jnp.pad(tok_embedding[TAIL0:, :].T,
                 ((0, 0), (0, 128 - (NEMB - TAIL0))))
  o = _embed(ids_in, tabT, tail)
  return o.transpose(2, 4, 0, 1, 3).reshape(BATCH, SEQ, DIM)
